# baseline (speedup 1.0000x reference)
import functools

import jax
import jax.numpy as jnp
from jax import lax
from jax.experimental import pallas as pl
from jax.experimental.pallas import tpu as pltpu
from jax.experimental.pallas import tpu_sc as plsc

N = 50000
D = 256
EPS = 1e-12
NC, NS, L = 2, 16, 16
NW = NC * NS
C = 16

SC_ROWS = 25600
SC_NITER = SC_ROWS // (NW * C)

TC_BLOCK = 400


def _vsqrt(x):
    i = plsc.bitcast(x, jnp.int32)
    i = jnp.int32(0x5F3759DF) - (i >> 1)
    y = plsc.bitcast(i, jnp.float32)
    for _ in range(3):
        y = y * (1.5 - (0.5 * x * y) * y)
    return x * y


def _sc_pair_loss(na, nb, dab):
    sa = jnp.maximum(_vsqrt(na), EPS)
    sb = jnp.maximum(_vsqrt(nb), EPS)
    return na / (sa * sa) + nb / (sb * sb) - 2.0 * (dab / (sa * sb))


_mesh = plsc.VectorSubcoreMesh(core_axis_name="c", subcore_axis_name="s")


@functools.partial(
    pl.kernel,
    mesh=_mesh,
    out_type=jax.ShapeDtypeStruct((NW, L), jnp.float32),
    scratch_types=[pltpu.VMEM((C, D), jnp.float32) for _ in range(8)]
    + [
        pltpu.VMEM((L,), jnp.float32),
        pltpu.SemaphoreType.DMA,
        pltpu.SemaphoreType.DMA,
    ],
    compiler_params=pltpu.CompilerParams(
        use_tc_tiling_on_sc=True, needs_layout_passes=False
    ),
)
def _sc_loss(p1, q2, p2, q1, out, b0, b1, b2, b3, b4, b5, b6, b7, lbuf, s0, s1):
    wid = lax.axis_index("s") * NC + lax.axis_index("c")
    row_iota = lax.iota(jnp.int32, L)
    zero = jnp.zeros((L,), jnp.float32)
    srcs = (p1, q2, p2, q1)
    bufs = ((b0, b1, b2, b3), (b4, b5, b6, b7))
    sems = (s0, s1)

    def start(i, slot):
        base = (wid + i * NW) * C
        for src, dst in zip(srcs, bufs[slot]):
            pltpu.async_copy(src.at[pl.ds(base, C)], dst, sems[slot])

    def wait(i, slot):
        base = (wid + i * NW) * C
        for src, dst in zip(srcs, bufs[slot]):
            pltpu.make_async_copy(src.at[pl.ds(base, C)], dst, sems[slot]).wait()

    def compute(slot, loss):
        c1, c2, c3, c4 = bufs[slot]

        def d_body(d, accs):
            na1, nb1, dd1, na2, nb2, dd2 = accs
            col = lax.full((L,), d, jnp.int32)
            a = plsc.load_gather(c1, [row_iota, col])
            b = plsc.load_gather(c2, [row_iota, col])
            c = plsc.load_gather(c3, [row_iota, col])
            e = plsc.load_gather(c4, [row_iota, col])
            return (
                na1 + a * a,
                nb1 + b * b,
                dd1 + a * b,
                na2 + c * c,
                nb2 + e * e,
                dd2 + c * e,
            )

        na1, nb1, dd1, na2, nb2, dd2 = lax.fori_loop(
            0, D, d_body, (zero, zero, zero, zero, zero, zero), unroll=8
        )
        return loss + _sc_pair_loss(na1, nb1, dd1) + _sc_pair_loss(na2, nb2, dd2)

    start(0, 0)

    def body(j, loss):
        for k in (0, 1):
            i = 2 * j + k

            @pl.when(i + 1 < SC_NITER)
            def _():
                start(i + 1, 1 - k)

            wait(i, k)
            loss = compute(k, loss)
        return loss

    lbuf[...] = lax.fori_loop(0, SC_NITER // 2, body, zero)
    pltpu.sync_copy(lbuf, out.at[wid])


def _tc_row_terms(a, b):
    na = jnp.sum(a * a, axis=1)
    nb = jnp.sum(b * b, axis=1)
    dab = jnp.sum(a * b, axis=1)
    sa = jnp.maximum(jnp.sqrt(na), EPS)
    sb = jnp.maximum(jnp.sqrt(nb), EPS)
    return jnp.sum(na / (sa * sa) + nb / (sb * sb) - 2.0 * dab / (sa * sb))


def _tc_body(p1_ref, q2_ref, p2_ref, q1_ref, out_ref):
    i = pl.program_id(0)

    partial = _tc_row_terms(p1_ref[...], q2_ref[...]) + _tc_row_terms(
        p2_ref[...], q1_ref[...]
    )

    @pl.when(i == 0)
    def _():
        out_ref[0, 0] = 0.0

    out_ref[0, 0] += partial


@jax.jit
def kernel(pred1, proj2, pred2, proj1):
    sc_partials = _sc_loss(pred1, proj2, pred2, proj1)

    tc_rows = N - SC_ROWS
    off = SC_ROWS // TC_BLOCK
    spec = pl.BlockSpec((TC_BLOCK, D), lambda i: (off + i, 0))
    tc_out = pl.pallas_call(
        _tc_body,
        grid=(tc_rows // TC_BLOCK,),
        in_specs=[spec, spec, spec, spec],
        out_specs=pl.BlockSpec((1, 1), lambda i: (0, 0), memory_space=pltpu.SMEM),
        out_shape=jax.ShapeDtypeStruct((1, 1), jnp.float32),
    )(pred1, proj2, pred2, proj1)

    return (jnp.sum(sc_partials) + tc_out[0, 0]) / 2.0


# speedup vs baseline: 4.3041x; 4.3041x over previous
import functools

import jax
import jax.numpy as jnp
from jax import lax
from jax.experimental import pallas as pl
from jax.experimental.pallas import tpu as pltpu
from jax.experimental.pallas import tpu_sc as plsc

N = 50000
D = 256
EPS = 1e-12
NC, NS, L = 2, 16, 16
NW = NC * NS
C = 16

SC_ROWS = 25600
SC_NITER = SC_ROWS // (NW * C)

TC_BLOCK = 400


def _vsqrt(x):
    i = plsc.bitcast(x, jnp.int32)
    i = jnp.int32(0x5F3759DF) - (i >> 1)
    y = plsc.bitcast(i, jnp.float32)
    for _ in range(3):
        y = y * (1.5 - (0.5 * x * y) * y)
    return x * y


def _sc_pair_loss(na, nb, dab):
    sa = jnp.maximum(_vsqrt(na), EPS)
    sb = jnp.maximum(_vsqrt(nb), EPS)
    return na / (sa * sa) + nb / (sb * sb) - 2.0 * (dab / (sa * sb))


_mesh = plsc.VectorSubcoreMesh(core_axis_name="c", subcore_axis_name="s")


@functools.partial(
    pl.kernel,
    mesh=_mesh,
    out_type=jax.ShapeDtypeStruct((NW, L), jnp.float32),
    scratch_types=[pltpu.VMEM((C, D), jnp.float32) for _ in range(8)]
    + [pltpu.VMEM((C * 17,), jnp.float32) for _ in range(6)]
    + [
        pltpu.VMEM((L,), jnp.float32),
        pltpu.SemaphoreType.DMA,
        pltpu.SemaphoreType.DMA,
    ],
    compiler_params=pltpu.CompilerParams(
        use_tc_tiling_on_sc=True, needs_layout_passes=False
    ),
)
def _sc_loss(
    p1, q2, p2, q1, out,
    b0, b1, b2, b3, b4, b5, b6, b7,
    t0, t1, t2, t3, t4, t5,
    lbuf, s0, s1,
):
    wid = lax.axis_index("s") * NC + lax.axis_index("c")
    row_iota = lax.iota(jnp.int32, L)
    zero = jnp.zeros((L,), jnp.float32)
    srcs = (p1, q2, p2, q1)
    bufs = ((b0, b1, b2, b3), (b4, b5, b6, b7))
    tr = (t0, t1, t2, t3, t4, t5)
    sems = (s0, s1)

    def start(i, slot):
        base = (wid + i * NW) * C
        for src, dst in zip(srcs, bufs[slot]):
            pltpu.async_copy(src.at[pl.ds(base, C)], dst, sems[slot])

    def wait(i, slot):
        base = (wid + i * NW) * C
        for src, dst in zip(srcs, bufs[slot]):
            pltpu.make_async_copy(src.at[pl.ds(base, C)], dst, sems[slot]).wait()

    def compute(slot, loss):
        c1, c2, c3, c4 = bufs[slot]

        def r_body(r, carry):
            na1 = nb1 = dd1 = na2 = nb2 = dd2 = zero
            for k in range(D // L):
                sl = pl.ds(k * L, L)
                a = c1[r, sl]
                b = c2[r, sl]
                c = c3[r, sl]
                e = c4[r, sl]
                na1 += a * a
                nb1 += b * b
                dd1 += a * b
                na2 += c * c
                nb2 += e * e
                dd2 += c * e
            off = pl.ds(r * 17, L)
            t0[off] = na1
            t1[off] = nb1
            t2[off] = dd1
            t3[off] = na2
            t4[off] = nb2
            t5[off] = dd2
            return carry

        lax.fori_loop(0, C, r_body, 0, unroll=2)

        sums = []
        for t in tr:
            acc = zero
            for k in range(D // L):
                acc += plsc.load_gather(t, [row_iota * 17 + k])
            sums.append(acc)
        na1, nb1, dd1, na2, nb2, dd2 = sums
        return loss + _sc_pair_loss(na1, nb1, dd1) + _sc_pair_loss(na2, nb2, dd2)

    start(0, 0)

    def body(j, loss):
        for k in (0, 1):
            i = 2 * j + k

            @pl.when(i + 1 < SC_NITER)
            def _():
                start(i + 1, 1 - k)

            wait(i, k)
            loss = compute(k, loss)
        return loss

    lbuf[...] = lax.fori_loop(0, SC_NITER // 2, body, zero)
    pltpu.sync_copy(lbuf, out.at[wid])


def _tc_row_terms(a, b):
    na = jnp.sum(a * a, axis=1)
    nb = jnp.sum(b * b, axis=1)
    dab = jnp.sum(a * b, axis=1)
    sa = jnp.maximum(jnp.sqrt(na), EPS)
    sb = jnp.maximum(jnp.sqrt(nb), EPS)
    return jnp.sum(na / (sa * sa) + nb / (sb * sb) - 2.0 * dab / (sa * sb))


def _tc_body(p1_ref, q2_ref, p2_ref, q1_ref, out_ref):
    i = pl.program_id(0)

    partial = _tc_row_terms(p1_ref[...], q2_ref[...]) + _tc_row_terms(
        p2_ref[...], q1_ref[...]
    )

    @pl.when(i == 0)
    def _():
        out_ref[0, 0] = 0.0

    out_ref[0, 0] += partial


@jax.jit
def kernel(pred1, proj2, pred2, proj1):
    sc_partials = _sc_loss(pred1, proj2, pred2, proj1)

    tc_rows = N - SC_ROWS
    off = SC_ROWS // TC_BLOCK
    spec = pl.BlockSpec((TC_BLOCK, D), lambda i: (off + i, 0))
    tc_out = pl.pallas_call(
        _tc_body,
        grid=(tc_rows // TC_BLOCK,),
        in_specs=[spec, spec, spec, spec],
        out_specs=pl.BlockSpec((1, 1), lambda i: (0, 0), memory_space=pltpu.SMEM),
        out_shape=jax.ShapeDtypeStruct((1, 1), jnp.float32),
    )(pred1, proj2, pred2, proj1)

    return (jnp.sum(sc_partials) + tc_out[0, 0]) / 2.0


# speedup vs baseline: 4.6946x; 1.0907x over previous
import functools

import jax
import jax.numpy as jnp
from jax import lax
from jax.experimental import pallas as pl
from jax.experimental.pallas import tpu as pltpu
from jax.experimental.pallas import tpu_sc as plsc

N = 50000
D = 256
EPS = 1e-12
NC, NS, L = 2, 16, 16
NW = NC * NS
C = 16

SC_ROWS = 16384
SC_NITER = SC_ROWS // (NW * C)
TC_ROWS = N - SC_ROWS

TC_BLOCK = 3056


def _vsqrt(x):
    i = plsc.bitcast(x, jnp.int32)
    i = jnp.int32(0x5F3759DF) - (i >> 1)
    y = plsc.bitcast(i, jnp.float32)
    for _ in range(3):
        y = y * (1.5 - (0.5 * x * y) * y)
    return x * y


def _sc_pair_loss(na, nb, dab):
    sa = jnp.maximum(_vsqrt(na), EPS)
    sb = jnp.maximum(_vsqrt(nb), EPS)
    return na / (sa * sa) + nb / (sb * sb) - 2.0 * (dab / (sa * sb))


_mesh = plsc.VectorSubcoreMesh(core_axis_name="c", subcore_axis_name="s")


@functools.partial(
    pl.kernel,
    mesh=_mesh,
    out_type=jax.ShapeDtypeStruct((NW, L), jnp.float32),
    scratch_types=[pltpu.VMEM((C, D), jnp.float32) for _ in range(8)]
    + [pltpu.VMEM((C * 17,), jnp.float32) for _ in range(6)]
    + [
        pltpu.VMEM((L,), jnp.float32),
        pltpu.SemaphoreType.DMA,
        pltpu.SemaphoreType.DMA,
    ],
    compiler_params=pltpu.CompilerParams(
        use_tc_tiling_on_sc=True, needs_layout_passes=False
    ),
)
def _sc_loss(
    p1, q2, p2, q1, out,
    b0, b1, b2, b3, b4, b5, b6, b7,
    t0, t1, t2, t3, t4, t5,
    lbuf, s0, s1,
):
    wid = lax.axis_index("s") * NC + lax.axis_index("c")
    row_iota = lax.iota(jnp.int32, L)
    zero = jnp.zeros((L,), jnp.float32)
    srcs = (p1, q2, p2, q1)
    bufs = ((b0, b1, b2, b3), (b4, b5, b6, b7))
    tr = (t0, t1, t2, t3, t4, t5)
    sems = (s0, s1)

    def start(i, slot):
        base = TC_ROWS + (wid + i * NW) * C
        for src, dst in zip(srcs, bufs[slot]):
            pltpu.async_copy(src.at[pl.ds(base, C)], dst, sems[slot])

    def wait(i, slot):
        base = TC_ROWS + (wid + i * NW) * C
        for src, dst in zip(srcs, bufs[slot]):
            pltpu.make_async_copy(src.at[pl.ds(base, C)], dst, sems[slot]).wait()

    def compute(slot, loss):
        c1, c2, c3, c4 = bufs[slot]

        def r_body(r, carry):
            na1 = nb1 = dd1 = na2 = nb2 = dd2 = zero
            for k in range(D // L):
                sl = pl.ds(k * L, L)
                a = c1[r, sl]
                b = c2[r, sl]
                c = c3[r, sl]
                e = c4[r, sl]
                na1 += a * a
                nb1 += b * b
                dd1 += a * b
                na2 += c * c
                nb2 += e * e
                dd2 += c * e
            off = pl.ds(r * 17, L)
            t0[off] = na1
            t1[off] = nb1
            t2[off] = dd1
            t3[off] = na2
            t4[off] = nb2
            t5[off] = dd2
            return carry

        lax.fori_loop(0, C, r_body, 0, unroll=2)

        sums = []
        for t in tr:
            acc = zero
            for k in range(D // L):
                acc += plsc.load_gather(t, [row_iota * 17 + k])
            sums.append(acc)
        na1, nb1, dd1, na2, nb2, dd2 = sums
        return loss + _sc_pair_loss(na1, nb1, dd1) + _sc_pair_loss(na2, nb2, dd2)

    start(0, 0)

    def body(j, loss):
        for k in (0, 1):
            i = 2 * j + k

            @pl.when(i + 1 < SC_NITER)
            def _():
                start(i + 1, 1 - k)

            wait(i, k)
            loss = compute(k, loss)
        return loss

    lbuf[...] = lax.fori_loop(0, SC_NITER // 2, body, zero)
    pltpu.sync_copy(lbuf, out.at[wid])


def _tc_row_terms(a, b):
    na = jnp.sum(a * a, axis=1)
    nb = jnp.sum(b * b, axis=1)
    dab = jnp.sum(a * b, axis=1)
    sa = jnp.maximum(jnp.sqrt(na), EPS)
    sb = jnp.maximum(jnp.sqrt(nb), EPS)
    return jnp.sum(na / (sa * sa) + nb / (sb * sb) - 2.0 * dab / (sa * sb))


def _tc_body(p1_ref, q2_ref, p2_ref, q1_ref, out_ref):
    i = pl.program_id(0)

    partial = _tc_row_terms(p1_ref[...], q2_ref[...]) + _tc_row_terms(
        p2_ref[...], q1_ref[...]
    )

    @pl.when(i == 0)
    def _():
        out_ref[0, 0] = 0.0

    out_ref[0, 0] += partial


@jax.jit
def kernel(pred1, proj2, pred2, proj1):
    sc_partials = _sc_loss(pred1, proj2, pred2, proj1)

    spec = pl.BlockSpec((TC_BLOCK, D), lambda i: (i, 0))
    tc_out = pl.pallas_call(
        _tc_body,
        grid=(TC_ROWS // TC_BLOCK,),
        in_specs=[spec, spec, spec, spec],
        out_specs=pl.BlockSpec((1, 1), lambda i: (0, 0), memory_space=pltpu.SMEM),
        out_shape=jax.ShapeDtypeStruct((1, 1), jnp.float32),
    )(pred1, proj2, pred2, proj1)

    return (jnp.sum(sc_partials) + tc_out[0, 0]) / 2.0
